# baseline (speedup 1.0000x reference)
import functools

import jax
import jax.numpy as jnp
from jax import lax
from jax.experimental import pallas as pl
from jax.experimental.pallas import tpu as pltpu

_VMEM_LIMIT = 48 * 1024 * 1024


def _f32dot(a, b):
    return jnp.dot(a, b, preferred_element_type=jnp.float32)


def _hidot(a, b):
    return jnp.dot(a, b, precision=jax.lax.Precision.HIGHEST,
                   preferred_element_type=jnp.float32)


def _pass_body(n_acc, setcomp, n_tiles, v, te, f, *refs):
    nfT_ref = refs[0]
    accs = refs[1:1 + n_acc]
    (efT_ref, srcR_ref, dstC_ref, wiT_ref, gw_ref, ew_ref, bm_ref,
     wnh_ref, wnv_ref, bn_ref, out_ref, ptab_ref, acc_ref,
     msg_ref) = refs[1 + n_acc:]

    t = pl.program_id(1)

    @pl.when(t == 0)
    def _init():
        h = jnp.maximum(_hidot(wiT_ref[...], nfT_ref[...]), 0.0)
        for a in accs:
            hv = a[:f, :] + a[f:, :]
            h = jnp.maximum(_hidot(wnh_ref[...], h)
                            + _hidot(wnv_ref[...], hv) + bn_ref[...], 0.0)
        ptab_ref[...] = _hidot(gw_ref[...], h).astype(jnp.bfloat16)
        acc_ref[...] = jnp.zeros_like(acc_ref)
        msg_ref[...] = jnp.zeros_like(msg_ref)


    dc = dstC_ref[...].astype(jnp.int16)
    ohS = jnp.where(lax.broadcasted_iota(jnp.int16, (te, v), 1) == dc,
                    jnp.bfloat16(1), jnp.bfloat16(0))
    acc_ref[...] += _f32dot(msg_ref[...], ohS)

    sr = srcR_ref[...].astype(jnp.int16)
    ohG = jnp.where(lax.broadcasted_iota(jnp.int16, (v, te), 0) == sr,
                    jnp.bfloat16(1), jnp.bfloat16(0))
    heT = _f32dot(ptab_ref[...], ohG)

    cT = _f32dot(ew_ref[...], efT_ref[...])
    if setcomp:
        msg_ref[...] = (heT * cT).astype(jnp.bfloat16)
    else:
        msg_ref[...] = jnp.maximum(heT + cT + bm_ref[...], 0.0
                                   ).astype(jnp.bfloat16)

    @pl.when(t == n_tiles)
    def _finish():
        out_ref[...] = acc_ref[...]


def _edge_pass(n_acc, setcomp, te, nfT, accs, efT, srcR, dstC, weights):
    _, v = nfT.shape
    f = weights[1].shape[0]
    e = srcR.shape[1]
    n_tiles = e // (2 * te)
    const = lambda c, t: (0, 0)

    last = n_tiles - 1
    in_specs = [pl.BlockSpec((nfT.shape[0], v), const)]
    in_specs += [pl.BlockSpec((2 * f, v), const)] * n_acc
    in_specs += [
        pl.BlockSpec((efT.shape[0], te),
                     lambda c, t: (0, c * n_tiles + jnp.minimum(t, last))),
        pl.BlockSpec((1, te),
                     lambda c, t: (0, c * n_tiles + jnp.minimum(t, last))),
        pl.BlockSpec((te, 1),
                     lambda c, t: (c * n_tiles + jnp.maximum(t - 1, 0), 0)),
    ]
    in_specs += [pl.BlockSpec(w.shape, const) for w in weights]

    body = functools.partial(_pass_body, n_acc, setcomp, n_tiles, v, te, f)
    return pl.pallas_call(
        body,
        out_shape=jax.ShapeDtypeStruct((2 * f, v), jnp.float32),
        grid=(2, n_tiles + 1),
        in_specs=in_specs,
        out_specs=pl.BlockSpec((f, v), lambda c, t: (c, 0)),
        scratch_shapes=[pltpu.VMEM((f, v), jnp.bfloat16),
                        pltpu.VMEM((f, v), jnp.float32),
                        pltpu.VMEM((f, te), jnp.bfloat16)],
        compiler_params=pltpu.CompilerParams(
            dimension_semantics=("parallel", "arbitrary"),
            vmem_limit_bytes=_VMEM_LIMIT),
    )(nfT, *accs, efT, srcR, dstC, *weights)


def _final_body(f, nfT_ref, a1, a2, a3, a4, wiT_ref, wnh_ref, wnv_ref,
                bn_ref, wsT_ref, out_ref):
    h = jnp.maximum(_hidot(wiT_ref[...], nfT_ref[...]), 0.0)
    for a in (a1, a2, a3):
        hv = a[:f, :] + a[f:, :]
        h = jnp.maximum(_hidot(wnh_ref[...], h)
                        + _hidot(wnv_ref[...], hv) + bn_ref[...], 0.0)
    nbr = a4[:f, :] + a4[f:, :]
    out_ref[...] = nbr * _hidot(wsT_ref[...], h)


def _final_call(nfT, acc1, acc2, acc3, acc4, weights):
    k, v = nfT.shape
    f = weights[1].shape[0]
    vb = v // 2
    const = lambda c: (0, 0)
    in_specs = [pl.BlockSpec((k, vb), lambda c: (0, c))]
    in_specs += [pl.BlockSpec((2 * f, vb), lambda c: (0, c))] * 4
    in_specs += [pl.BlockSpec(w.shape, const) for w in weights]
    return pl.pallas_call(
        functools.partial(_final_body, f),
        out_shape=jax.ShapeDtypeStruct((f, v), jnp.float32),
        grid=(2,),
        in_specs=in_specs,
        out_specs=pl.BlockSpec((f, vb), lambda c: (0, c)),
        compiler_params=pltpu.CompilerParams(
            dimension_semantics=("parallel",),
            vmem_limit_bytes=_VMEM_LIMIT),
    )(nfT, acc1, acc2, acc3, acc4, *weights)


def kernel(node_feats, edge_feats, src, dst, w_in, w_msg_h, w_msg_e, b_msg,
           w_new_h, w_new_v, b_new, w_edge, w_node, w_self, *, edge_tile=256):
    v = node_feats.shape[0]
    e = src.shape[0]
    f = w_in.shape[1]
    assert e % (2 * edge_tile) == 0 and v % 256 == 0

    nfT = node_feats.astype(jnp.float32).T
    efT = edge_feats.astype(jnp.float32).T.astype(jnp.bfloat16)
    srcR = src.astype(jnp.int32).reshape(1, e)
    dstC = dst.astype(jnp.int32).reshape(e, 1)

    wiT = w_in.astype(jnp.float32).T
    wmhT = w_msg_h.astype(jnp.float32).T
    wmeT = w_msg_e.astype(jnp.float32).T.astype(jnp.bfloat16)
    bmT = b_msg.astype(jnp.float32).T
    wnhT = w_new_h.astype(jnp.float32).T
    wnvT = w_new_v.astype(jnp.float32).T
    bnT = b_new.astype(jnp.float32).T
    wedT = w_edge.astype(jnp.float32).T.astype(jnp.bfloat16)
    wndT = w_node.astype(jnp.float32).T
    wsT = w_self.astype(jnp.float32).T

    layer_w = (wiT, wmhT, wmeT, bmT, wnhT, wnvT, bnT)
    sc_w = (wiT, wndT, wedT, bmT, wnhT, wnvT, bnT)

    acc1 = _edge_pass(0, False, edge_tile, nfT, (), efT, srcR, dstC, layer_w)
    acc2 = _edge_pass(1, False, edge_tile, nfT, (acc1,), efT, srcR, dstC,
                      layer_w)
    acc3 = _edge_pass(2, False, edge_tile, nfT, (acc1, acc2), efT, srcR,
                      dstC, layer_w)
    acc4 = _edge_pass(3, True, edge_tile, nfT, (acc1, acc2, acc3), efT,
                      srcR, dstC, sc_w)
    outT = _final_call(nfT, acc1, acc2, acc3, acc4,
                       (wiT, wnhT, wnvT, bnT, wsT))
    return outT.T


# speedup vs baseline: 1.2016x; 1.2016x over previous
import functools

import jax
import jax.numpy as jnp
from jax import lax
from jax.experimental import pallas as pl
from jax.experimental.pallas import tpu as pltpu

_VMEM_LIMIT = 48 * 1024 * 1024


def _f32dot(a, b):
    return jnp.dot(a, b, preferred_element_type=jnp.float32)


def _hidot(a, b):
    return jnp.dot(a, b, precision=jax.lax.Precision.HIGHEST,
                   preferred_element_type=jnp.float32)


def _pass_body(n_acc, setcomp, n_tiles, v, te, f, *refs):
    nfT_ref = refs[0]
    accs = refs[1:1 + n_acc]
    (efT_ref, srcR_ref, dstC_ref, wiT_ref, gw_ref, ew_ref, bm_ref,
     wnh_ref, wnv_ref, bn_ref, out_ref, ptab_ref, acc_ref,
     msg_ref) = refs[1 + n_acc:]

    t = pl.program_id(1)

    @pl.when(t == 0)
    def _init():
        h = jnp.maximum(_hidot(wiT_ref[...], nfT_ref[...]), 0.0)
        for a in accs:
            hv = a[:f, :] + a[f:, :]
            h = jnp.maximum(_hidot(wnh_ref[...], h)
                            + _hidot(wnv_ref[...], hv) + bn_ref[...], 0.0)
        ptab_ref[...] = _hidot(gw_ref[...], h).astype(jnp.bfloat16)
        acc_ref[...] = jnp.zeros_like(acc_ref)
        msg_ref[...] = jnp.zeros_like(msg_ref)


    dc = dstC_ref[...].astype(jnp.int16)
    ohS = jnp.where(lax.broadcasted_iota(jnp.int16, (te, v), 1) == dc,
                    jnp.bfloat16(1), jnp.bfloat16(0))
    acc_ref[...] += _f32dot(msg_ref[...], ohS)

    sr = srcR_ref[...].astype(jnp.int16)
    ohG = jnp.where(lax.broadcasted_iota(jnp.int16, (v, te), 0) == sr,
                    jnp.bfloat16(1), jnp.bfloat16(0))
    heT = _f32dot(ptab_ref[...], ohG)

    cT = _f32dot(ew_ref[...], efT_ref[0])
    if setcomp:
        msg_ref[...] = (heT * cT).astype(jnp.bfloat16)
    else:
        msg_ref[...] = jnp.maximum(heT + cT + bm_ref[...], 0.0
                                   ).astype(jnp.bfloat16)

    @pl.when(t == n_tiles)
    def _finish():
        out_ref[...] = acc_ref[...]


def _edge_pass(n_acc, setcomp, te, nfT, accs, efT, srcR, dstC, weights):
    _, v = nfT.shape
    f = weights[1].shape[0]
    e = srcR.shape[1]
    assert efT.shape == (e // te, 16, te)
    n_tiles = e // (2 * te)
    const = lambda c, t: (0, 0)

    last = n_tiles - 1
    in_specs = [pl.BlockSpec((nfT.shape[0], v), const)]
    in_specs += [pl.BlockSpec((2 * f, v), const)] * n_acc
    in_specs += [
        pl.BlockSpec((1, efT.shape[1], te),
                     lambda c, t: (c * n_tiles + jnp.minimum(t, last), 0, 0)),
        pl.BlockSpec((1, te),
                     lambda c, t: (0, c * n_tiles + jnp.minimum(t, last))),
        pl.BlockSpec((te, 1),
                     lambda c, t: (c * n_tiles + jnp.maximum(t - 1, 0), 0)),
    ]
    in_specs += [pl.BlockSpec(w.shape, const) for w in weights]

    body = functools.partial(_pass_body, n_acc, setcomp, n_tiles, v, te, f)
    return pl.pallas_call(
        body,
        out_shape=jax.ShapeDtypeStruct((2 * f, v), jnp.float32),
        grid=(2, n_tiles + 1),
        in_specs=in_specs,
        out_specs=pl.BlockSpec((f, v), lambda c, t: (c, 0)),
        scratch_shapes=[pltpu.VMEM((f, v), jnp.bfloat16),
                        pltpu.VMEM((f, v), jnp.float32),
                        pltpu.VMEM((f, te), jnp.bfloat16)],
        compiler_params=pltpu.CompilerParams(
            dimension_semantics=("parallel", "arbitrary"),
            vmem_limit_bytes=_VMEM_LIMIT),
    )(nfT, *accs, efT, srcR, dstC, *weights)


def _final_body(f, nfT_ref, a1, a2, a3, a4, wiT_ref, wnh_ref, wnv_ref,
                bn_ref, wsT_ref, out_ref):
    h = jnp.maximum(_hidot(wiT_ref[...], nfT_ref[...]), 0.0)
    for a in (a1, a2, a3):
        hv = a[:f, :] + a[f:, :]
        h = jnp.maximum(_hidot(wnh_ref[...], h)
                        + _hidot(wnv_ref[...], hv) + bn_ref[...], 0.0)
    nbr = a4[:f, :] + a4[f:, :]
    out_ref[...] = nbr * _hidot(wsT_ref[...], h)


def _final_call(nfT, acc1, acc2, acc3, acc4, weights):
    k, v = nfT.shape
    f = weights[1].shape[0]
    vb = v // 2
    const = lambda c: (0, 0)
    in_specs = [pl.BlockSpec((k, vb), lambda c: (0, c))]
    in_specs += [pl.BlockSpec((2 * f, vb), lambda c: (0, c))] * 4
    in_specs += [pl.BlockSpec(w.shape, const) for w in weights]
    return pl.pallas_call(
        functools.partial(_final_body, f),
        out_shape=jax.ShapeDtypeStruct((f, v), jnp.float32),
        grid=(2,),
        in_specs=in_specs,
        out_specs=pl.BlockSpec((f, vb), lambda c: (0, c)),
        compiler_params=pltpu.CompilerParams(
            dimension_semantics=("parallel",),
            vmem_limit_bytes=_VMEM_LIMIT),
    )(nfT, acc1, acc2, acc3, acc4, *weights)


def kernel(node_feats, edge_feats, src, dst, w_in, w_msg_h, w_msg_e, b_msg,
           w_new_h, w_new_v, b_new, w_edge, w_node, w_self, *, edge_tile=512):
    v = node_feats.shape[0]
    e = src.shape[0]
    f = w_in.shape[1]
    assert e % (2 * edge_tile) == 0 and v % 256 == 0

    nfT = node_feats.astype(jnp.float32).T
    efT = (edge_feats.astype(jnp.float32).T.astype(jnp.bfloat16)
           .reshape(16, e // edge_tile, edge_tile).transpose(1, 0, 2))
    srcR = src.astype(jnp.int32).reshape(1, e)
    dstC = dst.astype(jnp.int32).reshape(e, 1)

    wiT = w_in.astype(jnp.float32).T
    wmhT = w_msg_h.astype(jnp.float32).T
    wmeT = w_msg_e.astype(jnp.float32).T.astype(jnp.bfloat16)
    bmT = b_msg.astype(jnp.float32).T
    wnhT = w_new_h.astype(jnp.float32).T
    wnvT = w_new_v.astype(jnp.float32).T
    bnT = b_new.astype(jnp.float32).T
    wedT = w_edge.astype(jnp.float32).T.astype(jnp.bfloat16)
    wndT = w_node.astype(jnp.float32).T
    wsT = w_self.astype(jnp.float32).T

    layer_w = (wiT, wmhT, wmeT, bmT, wnhT, wnvT, bnT)
    sc_w = (wiT, wndT, wedT, bmT, wnhT, wnvT, bnT)

    acc1 = _edge_pass(0, False, edge_tile, nfT, (), efT, srcR, dstC, layer_w)
    acc2 = _edge_pass(1, False, edge_tile, nfT, (acc1,), efT, srcR, dstC,
                      layer_w)
    acc3 = _edge_pass(2, False, edge_tile, nfT, (acc1, acc2), efT, srcR,
                      dstC, layer_w)
    acc4 = _edge_pass(3, True, edge_tile, nfT, (acc1, acc2, acc3), efT,
                      srcR, dstC, sc_w)
    outT = _final_call(nfT, acc1, acc2, acc3, acc4,
                       (wiT, wnhT, wnvT, bnT, wsT))
    return outT.T


# speedup vs baseline: 1.3943x; 1.1604x over previous
import functools

import jax
import jax.numpy as jnp
from jax import lax
from jax.experimental import pallas as pl
from jax.experimental.pallas import tpu as pltpu

_VMEM_LIMIT = 48 * 1024 * 1024


def _f32dot(a, b):
    return jnp.dot(a, b, preferred_element_type=jnp.float32)


def _hidot(a, b):
    return jnp.dot(a, b, precision=jax.lax.Precision.HIGHEST,
                   preferred_element_type=jnp.float32)


def _pass_body(project, setcomp, n_tiles, v, te, f,
               x_ref, efT_ref, srcR_ref, dstC_ref, wiT_ref, gw_ref, ew_ref,
               bm_ref, wnh_ref, wnv_ref, bn_ref, wsT_ref, out_ref,
               h_ref, ptab_ref, acc_ref, msg_ref):
    t = pl.program_id(0)

    @pl.when(t == 0)
    def _init():
        if project:
            h = jnp.maximum(_hidot(wiT_ref[...], x_ref[...]), 0.0)
        else:
            h = x_ref[...]
        h_ref[...] = h
        ptab_ref[...] = _hidot(gw_ref[...], h).astype(jnp.bfloat16)
        acc_ref[...] = jnp.zeros_like(acc_ref)
        msg_ref[...] = jnp.zeros_like(msg_ref)


    dc = dstC_ref[...].astype(jnp.int16)
    ohS = jnp.where(lax.broadcasted_iota(jnp.int16, (te, v), 1) == dc,
                    jnp.bfloat16(1), jnp.bfloat16(0))
    acc_ref[...] += _f32dot(msg_ref[...], ohS)

    sr = srcR_ref[...].astype(jnp.int16)
    ohG = jnp.where(lax.broadcasted_iota(jnp.int16, (v, te), 0) == sr,
                    jnp.bfloat16(1), jnp.bfloat16(0))
    heT = _f32dot(ptab_ref[...], ohG)

    cT = _f32dot(ew_ref[...], efT_ref[0])
    if setcomp:
        msg_ref[...] = (heT * cT).astype(jnp.bfloat16)
    else:
        msg_ref[...] = jnp.maximum(heT + cT + bm_ref[...], 0.0
                                   ).astype(jnp.bfloat16)

    @pl.when(t == n_tiles)
    def _finish():
        if setcomp:
            out_ref[...] = acc_ref[...] * _hidot(wsT_ref[...], h_ref[...])
        else:
            out_ref[...] = jnp.maximum(
                _hidot(wnh_ref[...], h_ref[...])
                + _hidot(wnv_ref[...], acc_ref[...]) + bn_ref[...], 0.0)


def _edge_pass(project, setcomp, te, x, efT, srcR, dstC, weights):
    f = weights[1].shape[0]
    v = x.shape[1]
    e = srcR.shape[1]
    n_tiles = e // te
    assert efT.shape == (n_tiles, 16, te)
    const = lambda t: (0, 0)

    last = n_tiles - 1
    in_specs = [
        pl.BlockSpec(x.shape, const),
        pl.BlockSpec((1, efT.shape[1], te),
                     lambda t: (jnp.minimum(t, last), 0, 0)),
        pl.BlockSpec((1, te), lambda t: (0, jnp.minimum(t, last))),
        pl.BlockSpec((te, 1), lambda t: (jnp.maximum(t - 1, 0), 0)),
    ]
    in_specs += [pl.BlockSpec(w.shape, const) for w in weights]

    body = functools.partial(_pass_body, project, setcomp, n_tiles, v, te, f)
    return pl.pallas_call(
        body,
        out_shape=jax.ShapeDtypeStruct((f, v), jnp.float32),
        grid=(n_tiles + 1,),
        in_specs=in_specs,
        out_specs=pl.BlockSpec((f, v), const),
        scratch_shapes=[pltpu.VMEM((f, v), jnp.float32),
                        pltpu.VMEM((f, v), jnp.bfloat16),
                        pltpu.VMEM((f, v), jnp.float32),
                        pltpu.VMEM((f, te), jnp.bfloat16)],
        compiler_params=pltpu.CompilerParams(
            dimension_semantics=("arbitrary",),
            vmem_limit_bytes=_VMEM_LIMIT),
    )(x, efT, srcR, dstC, *weights)


def kernel(node_feats, edge_feats, src, dst, w_in, w_msg_h, w_msg_e, b_msg,
           w_new_h, w_new_v, b_new, w_edge, w_node, w_self, *,
           edge_tile=1024):
    v = node_feats.shape[0]
    e = src.shape[0]
    assert e % edge_tile == 0 and v % 256 == 0

    nfT = node_feats.astype(jnp.float32).T
    efT = (edge_feats.astype(jnp.float32).T.astype(jnp.bfloat16)
           .reshape(16, e // edge_tile, edge_tile).transpose(1, 0, 2))
    srcR = src.astype(jnp.int32).reshape(1, e)
    dstC = dst.astype(jnp.int32).reshape(e, 1)

    wiT = w_in.astype(jnp.float32).T
    wmhT = w_msg_h.astype(jnp.float32).T
    wmeT = w_msg_e.astype(jnp.float32).T.astype(jnp.bfloat16)
    bmT = b_msg.astype(jnp.float32).T
    wnhT = w_new_h.astype(jnp.float32).T
    wnvT = w_new_v.astype(jnp.float32).T
    bnT = b_new.astype(jnp.float32).T
    wedT = w_edge.astype(jnp.float32).T.astype(jnp.bfloat16)
    wndT = w_node.astype(jnp.float32).T
    wsT = w_self.astype(jnp.float32).T

    layer_w = (wiT, wmhT, wmeT, bmT, wnhT, wnvT, bnT, wsT)
    sc_w = (wiT, wndT, wedT, bmT, wnhT, wnvT, bnT, wsT)

    h = _edge_pass(True, False, edge_tile, nfT, efT, srcR, dstC, layer_w)
    h = _edge_pass(False, False, edge_tile, h, efT, srcR, dstC, layer_w)
    h = _edge_pass(False, False, edge_tile, h, efT, srcR, dstC, layer_w)
    outT = _edge_pass(False, True, edge_tile, h, efT, srcR, dstC, sc_w)
    return outT.T


# speedup vs baseline: 1.4672x; 1.0522x over previous
import functools

import jax
import jax.numpy as jnp
from jax import lax
from jax.experimental import pallas as pl
from jax.experimental.pallas import tpu as pltpu

_VMEM_LIMIT = 56 * 1024 * 1024


def _f32dot(a, b):
    return jnp.dot(a, b, preferred_element_type=jnp.float32)


def _hidot(a, b):
    return jnp.dot(a, b, precision=jax.lax.Precision.HIGHEST,
                   preferred_element_type=jnp.float32)


def _pass_body(project, setcomp, n_tiles, v, te, f,
               x_ref, efT_ref, srcR_ref, dstC_ref, wiT_ref, gw_ref, ew_ref,
               bm_ref, wnh_ref, wnv_ref, bn_ref, wsT_ref, out_ref,
               h_ref, ptab_ref, acc_ref, msg_ref):
    t = pl.program_id(0)

    @pl.when(t == 0)
    def _init():
        if project:
            h = jnp.maximum(_hidot(wiT_ref[...], x_ref[...]), 0.0)
        else:
            h = x_ref[...]
        h_ref[...] = h
        ptab_ref[...] = _hidot(gw_ref[...], h).astype(jnp.bfloat16)
        acc_ref[...] = jnp.zeros_like(acc_ref)
        msg_ref[...] = jnp.zeros_like(msg_ref)


    dc = dstC_ref[...].astype(jnp.int16)
    ohS = jnp.where(lax.broadcasted_iota(jnp.int16, (te, v), 1) == dc,
                    jnp.bfloat16(1), jnp.bfloat16(0))
    acc_ref[...] += _f32dot(msg_ref[...], ohS)

    sr = srcR_ref[...].astype(jnp.int16)
    ohG = jnp.where(lax.broadcasted_iota(jnp.int16, (v, te), 0) == sr,
                    jnp.bfloat16(1), jnp.bfloat16(0))
    heT = _f32dot(ptab_ref[...], ohG)

    cT = _f32dot(ew_ref[...], efT_ref[0])
    if setcomp:
        msg_ref[...] = (heT * cT).astype(jnp.bfloat16)
    else:
        msg_ref[...] = jnp.maximum(heT + cT + bm_ref[...], 0.0
                                   ).astype(jnp.bfloat16)

    @pl.when(t == n_tiles)
    def _finish():
        if setcomp:
            out_ref[...] = acc_ref[...] * _hidot(wsT_ref[...], h_ref[...])
        else:
            out_ref[...] = jnp.maximum(
                _hidot(wnh_ref[...], h_ref[...])
                + _hidot(wnv_ref[...], acc_ref[...]) + bn_ref[...], 0.0)


def _edge_pass(project, setcomp, te, x, efT, srcR, dstC, weights):
    f = weights[1].shape[0]
    v = x.shape[1]
    e = srcR.shape[1]
    n_tiles = e // te
    assert efT.shape == (n_tiles, 16, te)
    const = lambda t: (0, 0)

    last = n_tiles - 1
    in_specs = [
        pl.BlockSpec(x.shape, const),
        pl.BlockSpec((1, efT.shape[1], te),
                     lambda t: (jnp.minimum(t, last), 0, 0)),
        pl.BlockSpec((1, te), lambda t: (0, jnp.minimum(t, last))),
        pl.BlockSpec((te, 1), lambda t: (jnp.maximum(t - 1, 0), 0)),
    ]
    in_specs += [pl.BlockSpec(w.shape, const) for w in weights]

    body = functools.partial(_pass_body, project, setcomp, n_tiles, v, te, f)
    return pl.pallas_call(
        body,
        out_shape=jax.ShapeDtypeStruct((f, v), jnp.float32),
        grid=(n_tiles + 1,),
        in_specs=in_specs,
        out_specs=pl.BlockSpec((f, v), const),
        scratch_shapes=[pltpu.VMEM((f, v), jnp.float32),
                        pltpu.VMEM((f, v), jnp.bfloat16),
                        pltpu.VMEM((f, v), jnp.float32),
                        pltpu.VMEM((f, te), jnp.bfloat16)],
        compiler_params=pltpu.CompilerParams(
            dimension_semantics=("arbitrary",),
            vmem_limit_bytes=_VMEM_LIMIT),
    )(x, efT, srcR, dstC, *weights)


def kernel(node_feats, edge_feats, src, dst, w_in, w_msg_h, w_msg_e, b_msg,
           w_new_h, w_new_v, b_new, w_edge, w_node, w_self, *,
           edge_tile=2048):
    v = node_feats.shape[0]
    e = src.shape[0]
    assert e % edge_tile == 0 and v % 256 == 0

    nfT = node_feats.astype(jnp.float32).T
    efT = (edge_feats.astype(jnp.float32).T.astype(jnp.bfloat16)
           .reshape(16, e // edge_tile, edge_tile).transpose(1, 0, 2))
    srcR = src.astype(jnp.int32).reshape(1, e)
    dstC = dst.astype(jnp.int32).reshape(e, 1)

    wiT = w_in.astype(jnp.float32).T
    wmhT = w_msg_h.astype(jnp.float32).T
    wmeT = w_msg_e.astype(jnp.float32).T.astype(jnp.bfloat16)
    bmT = b_msg.astype(jnp.float32).T
    wnhT = w_new_h.astype(jnp.float32).T
    wnvT = w_new_v.astype(jnp.float32).T
    bnT = b_new.astype(jnp.float32).T
    wedT = w_edge.astype(jnp.float32).T.astype(jnp.bfloat16)
    wndT = w_node.astype(jnp.float32).T
    wsT = w_self.astype(jnp.float32).T

    layer_w = (wiT, wmhT, wmeT, bmT, wnhT, wnvT, bnT, wsT)
    sc_w = (wiT, wndT, wedT, bmT, wnhT, wnvT, bnT, wsT)

    h = _edge_pass(True, False, edge_tile, nfT, efT, srcR, dstC, layer_w)
    h = _edge_pass(False, False, edge_tile, h, efT, srcR, dstC, layer_w)
    h = _edge_pass(False, False, edge_tile, h, efT, srcR, dstC, layer_w)
    outT = _edge_pass(False, True, edge_tile, h, efT, srcR, dstC, sc_w)
    return outT.T
